# Initial kernel scaffold; baseline (speedup 1.0000x reference)
import functools

import jax
import jax.numpy as jnp
from jax import lax
from jax.experimental import pallas as pl
from jax.experimental.pallas import tpu as pltpu
from jax.experimental.pallas import tpu_sc as plsc

N = 50000
E = 800000
G = 50
NGT = 30
EMBED = 16
H = 64

BLK = 1000
EBLK = 4000
SUB = 125
SB = 1000
NSUB = SB // SUB
TILES = 16
EPT = E // TILES
NSB = EPT // SB
RPT = N // TILES


def _embed_body(gate_ref, emb_ref, w1ha_ref, w1hb_ref, h0_ref, pa_ref, pb_ref):
    gate = gate_ref[...]
    onehot = (gate == lax.broadcasted_iota(jnp.int32, (BLK, NGT), 1)
              ).astype(jnp.float32)
    h0 = jnp.dot(onehot, emb_ref[...], preferred_element_type=jnp.float32)
    h0_ref[...] = h0
    pa_ref[...] = jnp.dot(h0, w1ha_ref[...], preferred_element_type=jnp.float32)
    pb_ref[...] = jnp.dot(h0, w1hb_ref[...], preferred_element_type=jnp.float32)


def _embed(gate2d, emb, w1ha, w1hb):
    return pl.pallas_call(
        _embed_body,
        grid=(N // BLK,),
        in_specs=[
            pl.BlockSpec((BLK, 1), lambda i: (i, 0)),
            pl.BlockSpec((NGT, EMBED), lambda i: (0, 0)),
            pl.BlockSpec((EMBED, 32), lambda i: (0, 0)),
            pl.BlockSpec((EMBED, 32), lambda i: (0, 0)),
        ],
        out_specs=[
            pl.BlockSpec((BLK, EMBED), lambda i: (i, 0)),
            pl.BlockSpec((BLK, 32), lambda i: (i, 0)),
            pl.BlockSpec((BLK, 32), lambda i: (i, 0)),
        ],
        out_shape=[
            jax.ShapeDtypeStruct((N, EMBED), jnp.float32),
            jax.ShapeDtypeStruct((N, 32), jnp.float32),
            jax.ShapeDtypeStruct((N, 32), jnp.float32),
        ],
    )(gate2d, emb, w1ha, w1hb)


def _r_body(ew_ref, *refs):
    ew = ew_ref[...]
    ws = refs[:6]
    bs = refs[6:12]
    outs = refs[12:]
    for k in range(6):
        outs[k][...] = (
            jnp.dot(ew, ws[k][...], preferred_element_type=jnp.float32)
            + bs[k][...]
        )


def _edge_r(edge_w, w_halves, b_halves):
    return pl.pallas_call(
        _r_body,
        grid=(E // EBLK,),
        in_specs=(
            [pl.BlockSpec((EBLK, 3), lambda i: (i, 0))]
            + [pl.BlockSpec((3, 32), lambda i: (0, 0))] * 6
            + [pl.BlockSpec((1, 32), lambda i: (0, 0))] * 6
        ),
        out_specs=[pl.BlockSpec((EBLK, 32), lambda i: (i, 0))] * 6,
        out_shape=[jax.ShapeDtypeStruct((E, 32), jnp.float32)] * 6,
    )(edge_w, *w_halves, *b_halves)


def _sc_body(src2d, dst2d, pa, pb, ra, rb, zrows, outa, outb,
             src_v, dst_v, prow, r_v, gsem, ssem, acc):
    c = lax.axis_index("c")
    s = lax.axis_index("s")

    pltpu.sync_copy(zrows, acc.at[pl.ds(s * RPT, RPT)])
    plsc.subcore_barrier()

    def half(p_hbm, r_hbm, out_hbm):
        def sb_body(b, carry):
            rowbase = s * (EPT // SUB) + b * NSUB
            ebase = s * EPT + b * SB
            pltpu.sync_copy(src2d.at[pl.ds(rowbase, NSUB)], src_v)
            pltpu.sync_copy(dst2d.at[pl.ds(rowbase, NSUB)], dst_v)
            pltpu.sync_copy(r_hbm.at[pl.ds(ebase, SB)], r_v)
            gh = [
                pltpu.async_copy(p_hbm.at[src_v.at[j]],
                                 prow.at[pl.ds(j * SUB, SUB)], gsem)
                for j in range(NSUB)
            ]
            for hdl in gh:
                hdl.wait()

            def ebody(i, carry2):
                for u in range(5):
                    k = i * 5 + u
                    for jj in range(2):
                        sl = pl.ds(jj * 16, 16)
                        prow[k, sl] = jnp.maximum(prow[k, sl] + r_v[k, sl], 0.0)
                return carry2

            lax.fori_loop(0, SB // 5, ebody, 0)
            sh = [
                pltpu.async_copy(prow.at[pl.ds(j * SUB, SUB)],
                                 acc.at[dst_v.at[j]], ssem, add=True)
                for j in range(NSUB)
            ]
            for hdl in sh:
                hdl.wait()
            return carry

        lax.fori_loop(0, NSB, sb_body, 0)
        plsc.subcore_barrier()
        pltpu.sync_copy(acc.at[pl.ds(s * RPT, RPT)],
                        out_hbm.at[pl.ds(s * RPT, RPT)])

    with pl.when(c == 0):
        half(pa, ra, outa)
    with pl.when(c == 1):
        half(pb, rb, outb)


_sc_edge = functools.partial(
    pl.kernel,
    out_type=(
        jax.ShapeDtypeStruct((N, 32), jnp.float32),
        jax.ShapeDtypeStruct((N, 32), jnp.float32),
    ),
    mesh=plsc.VectorSubcoreMesh(core_axis_name="c", subcore_axis_name="s"),
    scratch_types=[
        pltpu.VMEM((NSUB, SUB), jnp.int32),
        pltpu.VMEM((NSUB, SUB), jnp.int32),
        pltpu.VMEM((SB, 32), jnp.float32),
        pltpu.VMEM((SB, 32), jnp.float32),
        pltpu.SemaphoreType.DMA,
        pltpu.SemaphoreType.DMA,
        pltpu.VMEM_SHARED((N, 32), jnp.float32),
    ],
)(_sc_body)


def _make_node_body(in_dim, has_next):
    def body(*refs):
        (hp_ref, hna_ref, hnb_ref, gid_ref, w2a_ref, w2b0_ref, w2b1_ref) = refs[:7]
        idx = 7
        if has_next:
            w1na_ref, w1nb_ref = refs[idx:idx + 2]
            idx += 2
        h_ref = refs[idx]
        r_ref = refs[idx + 1]
        if has_next:
            pa_ref, pb_ref = refs[idx + 2:idx + 4]

        ht = (jnp.dot(hp_ref[...], w2a_ref[...], preferred_element_type=jnp.float32)
              + jnp.dot(hna_ref[...], w2b0_ref[...], preferred_element_type=jnp.float32)
              + jnp.dot(hnb_ref[...], w2b1_ref[...], preferred_element_type=jnp.float32))
        hl = jnp.maximum(ht, 0.0)
        ss = jnp.sum(hl * hl, axis=1, keepdims=True)
        nrm = jnp.maximum(jnp.sqrt(ss), 1e-12)
        h = hl / nrm
        h_ref[...] = h
        if has_next:
            pa_ref[...] = jnp.dot(h, w1na_ref[...], preferred_element_type=jnp.float32)
            pb_ref[...] = jnp.dot(h, w1nb_ref[...], preferred_element_type=jnp.float32)
        gid = gid_ref[...]
        onehot = (gid == lax.broadcasted_iota(jnp.int32, (BLK, G), 1)
                  ).astype(jnp.float32)
        contrib = lax.dot_general(onehot, h, (((0,), (0,)), ((), ())),
                                  preferred_element_type=jnp.float32)

        @pl.when(pl.program_id(0) == 0)
        def _():
            r_ref[...] = jnp.zeros_like(r_ref)

        r_ref[...] += contrib
    return body


def _node_update(h_prev, hna, hnb, gid2d, w2a, w2b0, w2b1, w1n=None):
    in_dim = h_prev.shape[1]
    has_next = w1n is not None
    in_specs = [
        pl.BlockSpec((BLK, in_dim), lambda i: (i, 0)),
        pl.BlockSpec((BLK, 32), lambda i: (i, 0)),
        pl.BlockSpec((BLK, 32), lambda i: (i, 0)),
        pl.BlockSpec((BLK, 1), lambda i: (i, 0)),
        pl.BlockSpec((in_dim, H), lambda i: (0, 0)),
        pl.BlockSpec((32, H), lambda i: (0, 0)),
        pl.BlockSpec((32, H), lambda i: (0, 0)),
    ]
    args = [h_prev, hna, hnb, gid2d, w2a, w2b0, w2b1]
    out_specs = [
        pl.BlockSpec((BLK, H), lambda i: (i, 0)),
        pl.BlockSpec((G, H), lambda i: (0, 0)),
    ]
    out_shape = [
        jax.ShapeDtypeStruct((N, H), jnp.float32),
        jax.ShapeDtypeStruct((G, H), jnp.float32),
    ]
    if has_next:
        in_specs += [pl.BlockSpec((H, 32), lambda i: (0, 0))] * 2
        args += [w1n[:, :32], w1n[:, 32:]]
        out_specs += [pl.BlockSpec((BLK, 32), lambda i: (i, 0))] * 2
        out_shape += [jax.ShapeDtypeStruct((N, 32), jnp.float32)] * 2
    return pl.pallas_call(
        _make_node_body(in_dim, has_next),
        grid=(N // BLK,),
        in_specs=in_specs,
        out_specs=out_specs,
        out_shape=out_shape,
    )(*args)


def kernel(gate_type, edge_index, edge_w, graph_id, params):
    layers = params['layers']
    emb = params['emb']

    gate2d = gate_type.reshape(N, 1)
    gid2d = graph_id.reshape(N, 1)
    src2d = edge_index[0].reshape(E // SUB, SUB)
    dst2d = edge_index[1].reshape(E // SUB, SUB)
    zrows = jnp.zeros((RPT, 32), jnp.float32)

    w_halves, b_halves = [], []
    dims = [EMBED, H, H]
    for l in range(3):
        w1 = layers[l]['W1']
        b1 = layers[l]['b1'].reshape(1, H)
        w_halves += [w1[dims[l]:, :32], w1[dims[l]:, 32:]]
        b_halves += [b1[:, :32], b1[:, 32:]]
    r_all = _edge_r(edge_w, w_halves, b_halves)

    w1_1 = layers[0]['W1']
    h0, pa, pb = _embed(gate2d, emb, w1_1[:EMBED, :32], w1_1[:EMBED, 32:])

    h = h0
    readouts = []
    for l in range(3):
        ra, rb = r_all[2 * l], r_all[2 * l + 1]
        hna, hnb = _sc_edge(src2d, dst2d, pa, pb, ra, rb, zrows)
        w2 = layers[l]['W2']
        in_dim = dims[l]
        w1n = layers[l + 1]['W1'][:H, :] if l < 2 else None
        outs = _node_update(h, hna, hnb, gid2d,
                            w2[:in_dim, :], w2[in_dim:in_dim + 32, :],
                            w2[in_dim + 32:, :], w1n)
        if l < 2:
            h, r_l, pa, pb = outs
        else:
            h, r_l = outs
        readouts.append(r_l)

    return h, jnp.concatenate(readouts, axis=1)


# speedup vs baseline: 2.7350x; 2.7350x over previous
import functools

import jax
import jax.numpy as jnp
from jax import lax
from jax.experimental import pallas as pl
from jax.experimental.pallas import tpu as pltpu
from jax.experimental.pallas import tpu_sc as plsc

N = 50000
E = 800000
G = 50
NGT = 30
EMBED = 16
H = 64

BLK = 1000
EBLK = 4000
SUB = 125
SB = 250
NSUB = SB // SUB
TILES = 16
EPT = E // TILES
NSB = EPT // SB
NP = 50048
RPT = NP // TILES


def _embed_body(gate_ref, emb_ref, w1ha_ref, w1hb_ref, h0_ref, pa_ref, pb_ref):
    gate = gate_ref[...]
    onehot = (gate == lax.broadcasted_iota(jnp.int32, (BLK, NGT), 1)
              ).astype(jnp.float32)
    h0 = jnp.dot(onehot, emb_ref[...], preferred_element_type=jnp.float32)
    h0_ref[...] = h0
    pa_ref[...] = jnp.dot(h0, w1ha_ref[...], preferred_element_type=jnp.float32)
    pb_ref[...] = jnp.dot(h0, w1hb_ref[...], preferred_element_type=jnp.float32)


def _embed(gate2d, emb, w1ha, w1hb):
    return pl.pallas_call(
        _embed_body,
        grid=(N // BLK,),
        in_specs=[
            pl.BlockSpec((BLK, 1), lambda i: (i, 0)),
            pl.BlockSpec((NGT, EMBED), lambda i: (0, 0)),
            pl.BlockSpec((EMBED, 32), lambda i: (0, 0)),
            pl.BlockSpec((EMBED, 32), lambda i: (0, 0)),
        ],
        out_specs=[
            pl.BlockSpec((BLK, EMBED), lambda i: (i, 0)),
            pl.BlockSpec((BLK, 32), lambda i: (i, 0)),
            pl.BlockSpec((BLK, 32), lambda i: (i, 0)),
        ],
        out_shape=[
            jax.ShapeDtypeStruct((N, EMBED), jnp.float32),
            jax.ShapeDtypeStruct((N, 32), jnp.float32),
            jax.ShapeDtypeStruct((N, 32), jnp.float32),
        ],
    )(gate2d, emb, w1ha, w1hb)


def _r_body(ew_ref, *refs):
    ew = ew_ref[...]
    ws = refs[:6]
    bs = refs[6:12]
    outs = refs[12:]
    for k in range(6):
        outs[k][...] = (
            jnp.dot(ew, ws[k][...], preferred_element_type=jnp.float32)
            + bs[k][...]
        )


def _edge_r(edge_w, w_halves, b_halves):
    return pl.pallas_call(
        _r_body,
        grid=(E // EBLK,),
        in_specs=(
            [pl.BlockSpec((EBLK, 3), lambda i: (i, 0))]
            + [pl.BlockSpec((3, 32), lambda i: (0, 0))] * 6
            + [pl.BlockSpec((1, 32), lambda i: (0, 0))] * 6
        ),
        out_specs=[pl.BlockSpec((EBLK, 32), lambda i: (i, 0))] * 6,
        out_shape=[jax.ShapeDtypeStruct((E, 32), jnp.float32)] * 6,
    )(edge_w, *w_halves, *b_halves)


def _sc_body(src2d, dst2d, pa, pb, ra, rb, zrows, outa, outb,
             src_v, dst_v, prow, r_v, gsem, ssem, acc):
    c = lax.axis_index("c")
    s = lax.axis_index("s")

    pltpu.sync_copy(zrows, acc.at[pl.ds(s * RPT, RPT)])
    plsc.subcore_barrier()

    def half(p_hbm, r_hbm, out_hbm):
        def sb_body(b, carry):
            rowbase = s * (EPT // SUB) + b * NSUB
            ebase = s * EPT + b * SB
            pltpu.sync_copy(src2d.at[pl.ds(rowbase, NSUB)], src_v)
            pltpu.sync_copy(dst2d.at[pl.ds(rowbase, NSUB)], dst_v)
            pltpu.sync_copy(r_hbm.at[pl.ds(ebase, SB)], r_v)
            gh = [
                pltpu.async_copy(p_hbm.at[src_v.at[j]],
                                 prow.at[pl.ds(j * SUB, SUB)], gsem)
                for j in range(NSUB)
            ]
            for hdl in gh:
                hdl.wait()

            def ebody(i, carry2):
                for u in range(5):
                    k = i * 5 + u
                    for jj in range(2):
                        sl = pl.ds(jj * 16, 16)
                        prow[k, sl] = jnp.maximum(prow[k, sl] + r_v[k, sl], 0.0)
                return carry2

            lax.fori_loop(0, SB // 5, ebody, 0)
            sh = [
                pltpu.async_copy(prow.at[pl.ds(j * SUB, SUB)],
                                 acc.at[dst_v.at[j]], ssem, add=True)
                for j in range(NSUB)
            ]
            for hdl in sh:
                hdl.wait()
            return carry

        lax.fori_loop(0, NSB, sb_body, 0)
        plsc.subcore_barrier()
        pltpu.sync_copy(acc.at[pl.ds(s * RPT, RPT)],
                        out_hbm.at[pl.ds(s * RPT, RPT)])

    @pl.when(c == 0)
    def _():
        half(pa, ra, outa)

    @pl.when(c == 1)
    def _():
        half(pb, rb, outb)


_sc_edge = functools.partial(
    pl.kernel,
    out_type=(
        jax.ShapeDtypeStruct((NP, 32), jnp.float32),
        jax.ShapeDtypeStruct((NP, 32), jnp.float32),
    ),
    mesh=plsc.VectorSubcoreMesh(core_axis_name="c", subcore_axis_name="s"),
    compiler_params=pltpu.CompilerParams(use_tc_tiling_on_sc=False),
    scratch_types=[
        pltpu.VMEM((NSUB, SUB), jnp.int32),
        pltpu.VMEM((NSUB, SUB), jnp.int32),
        pltpu.VMEM((SB, 32), jnp.float32),
        pltpu.VMEM((SB, 32), jnp.float32),
        pltpu.SemaphoreType.DMA,
        pltpu.SemaphoreType.DMA,
        pltpu.VMEM_SHARED((NP, 32), jnp.float32),
    ],
)(_sc_body)


def _make_node_body(in_dim, has_next):
    def body(*refs):
        (hp_ref, hna_ref, hnb_ref, gid_ref, w2a_ref, w2b0_ref, w2b1_ref) = refs[:7]
        idx = 7
        if has_next:
            w1na_ref, w1nb_ref = refs[idx:idx + 2]
            idx += 2
        h_ref = refs[idx]
        r_ref = refs[idx + 1]
        if has_next:
            pa_ref, pb_ref = refs[idx + 2:idx + 4]

        ht = (jnp.dot(hp_ref[...], w2a_ref[...], preferred_element_type=jnp.float32)
              + jnp.dot(hna_ref[...], w2b0_ref[...], preferred_element_type=jnp.float32)
              + jnp.dot(hnb_ref[...], w2b1_ref[...], preferred_element_type=jnp.float32))
        hl = jnp.maximum(ht, 0.0)
        ss = jnp.sum(hl * hl, axis=1, keepdims=True)
        nrm = jnp.maximum(jnp.sqrt(ss), 1e-12)
        h = hl / nrm
        h_ref[...] = h
        if has_next:
            pa_ref[...] = jnp.dot(h, w1na_ref[...], preferred_element_type=jnp.float32)
            pb_ref[...] = jnp.dot(h, w1nb_ref[...], preferred_element_type=jnp.float32)
        gid = gid_ref[...]
        onehot = (gid == lax.broadcasted_iota(jnp.int32, (BLK, G), 1)
                  ).astype(jnp.float32)
        contrib = lax.dot_general(onehot, h, (((0,), (0,)), ((), ())),
                                  preferred_element_type=jnp.float32)

        @pl.when(pl.program_id(0) == 0)
        def _():
            r_ref[...] = jnp.zeros_like(r_ref)

        r_ref[...] += contrib
    return body


def _node_update(h_prev, hna, hnb, gid2d, w2a, w2b0, w2b1, w1n=None):
    in_dim = h_prev.shape[1]
    has_next = w1n is not None
    in_specs = [
        pl.BlockSpec((BLK, in_dim), lambda i: (i, 0)),
        pl.BlockSpec((BLK, 32), lambda i: (i, 0)),
        pl.BlockSpec((BLK, 32), lambda i: (i, 0)),
        pl.BlockSpec((BLK, 1), lambda i: (i, 0)),
        pl.BlockSpec((in_dim, H), lambda i: (0, 0)),
        pl.BlockSpec((32, H), lambda i: (0, 0)),
        pl.BlockSpec((32, H), lambda i: (0, 0)),
    ]
    args = [h_prev, hna, hnb, gid2d, w2a, w2b0, w2b1]
    out_specs = [
        pl.BlockSpec((BLK, H), lambda i: (i, 0)),
        pl.BlockSpec((G, H), lambda i: (0, 0)),
    ]
    out_shape = [
        jax.ShapeDtypeStruct((N, H), jnp.float32),
        jax.ShapeDtypeStruct((G, H), jnp.float32),
    ]
    if has_next:
        in_specs += [pl.BlockSpec((H, 32), lambda i: (0, 0))] * 2
        args += [w1n[:, :32], w1n[:, 32:]]
        out_specs += [pl.BlockSpec((BLK, 32), lambda i: (i, 0))] * 2
        out_shape += [jax.ShapeDtypeStruct((N, 32), jnp.float32)] * 2
    return pl.pallas_call(
        _make_node_body(in_dim, has_next),
        grid=(N // BLK,),
        in_specs=in_specs,
        out_specs=out_specs,
        out_shape=out_shape,
    )(*args)


def kernel(gate_type, edge_index, edge_w, graph_id, params):
    layers = params['layers']
    emb = params['emb']

    gate2d = gate_type.reshape(N, 1)
    gid2d = graph_id.reshape(N, 1)
    src2d = edge_index[0].reshape(E // SUB, SUB)
    dst2d = edge_index[1].reshape(E // SUB, SUB)
    zrows = jnp.zeros((RPT, 32), jnp.float32)

    w_halves, b_halves = [], []
    dims = [EMBED, H, H]
    for l in range(3):
        w1 = layers[l]['W1']
        b1 = layers[l]['b1'].reshape(1, H)
        w_halves += [w1[dims[l]:, :32], w1[dims[l]:, 32:]]
        b_halves += [b1[:, :32], b1[:, 32:]]
    r_all = _edge_r(edge_w, w_halves, b_halves)

    w1_1 = layers[0]['W1']
    h0, pa, pb = _embed(gate2d, emb, w1_1[:EMBED, :32], w1_1[:EMBED, 32:])

    h = h0
    readouts = []
    for l in range(3):
        ra, rb = r_all[2 * l], r_all[2 * l + 1]
        hna, hnb = _sc_edge(src2d, dst2d, pa, pb, ra, rb, zrows)
        w2 = layers[l]['W2']
        in_dim = dims[l]
        w1n = layers[l + 1]['W1'][:H, :] if l < 2 else None
        outs = _node_update(h, hna, hnb, gid2d,
                            w2[:in_dim, :], w2[in_dim:in_dim + 32, :],
                            w2[in_dim + 32:, :], w1n)
        if l < 2:
            h, r_l, pa, pb = outs
        else:
            h, r_l = outs
        readouts.append(r_l)

    return h, jnp.concatenate(readouts, axis=1)


# speedup vs baseline: 4.0916x; 1.4960x over previous
import functools

import jax
import jax.numpy as jnp
from jax import lax
from jax.experimental import pallas as pl
from jax.experimental.pallas import tpu as pltpu
from jax.experimental.pallas import tpu_sc as plsc

N = 50000
E = 800000
G = 50
NGT = 30
EMBED = 16
H = 64

BLK = 1000
SUB = 100
SB = 400
NSUB = SB // SUB
GRP = SB // 16
TILES = 16
EPT = E // TILES
NSB = EPT // SB
NP = 50048
RPT = NP // TILES


def _embed_body(gate_ref, emb_ref, w1ha_ref, w1hb_ref, b1a_ref, b1b_ref,
                h0_ref, pa_ref, pb_ref):
    gate = gate_ref[...]
    onehot = (gate == lax.broadcasted_iota(jnp.int32, (BLK, NGT), 1)
              ).astype(jnp.float32)
    h0 = jnp.dot(onehot, emb_ref[...], preferred_element_type=jnp.float32)
    h0_ref[...] = h0
    pa_ref[...] = (jnp.dot(h0, w1ha_ref[...], preferred_element_type=jnp.float32)
                   + b1a_ref[...])
    pb_ref[...] = (jnp.dot(h0, w1hb_ref[...], preferred_element_type=jnp.float32)
                   + b1b_ref[...])


def _embed(gate2d, emb, w1ha, w1hb, b1a, b1b):
    return pl.pallas_call(
        _embed_body,
        grid=(N // BLK,),
        in_specs=[
            pl.BlockSpec((BLK, 1), lambda i: (i, 0)),
            pl.BlockSpec((NGT, EMBED), lambda i: (0, 0)),
            pl.BlockSpec((EMBED, 32), lambda i: (0, 0)),
            pl.BlockSpec((EMBED, 32), lambda i: (0, 0)),
            pl.BlockSpec((1, 32), lambda i: (0, 0)),
            pl.BlockSpec((1, 32), lambda i: (0, 0)),
        ],
        out_specs=[
            pl.BlockSpec((BLK, EMBED), lambda i: (i, 0)),
            pl.BlockSpec((BLK, 32), lambda i: (i, 0)),
            pl.BlockSpec((BLK, 32), lambda i: (i, 0)),
        ],
        out_shape=[
            jax.ShapeDtypeStruct((N, EMBED), jnp.float32),
            jax.ShapeDtypeStruct((N, 32), jnp.float32),
            jax.ShapeDtypeStruct((N, 32), jnp.float32),
        ],
    )(gate2d, emb, w1ha, w1hb, b1a, b1b)


def _sc_body(src2d, dst2d, pa, pb, ewg, wea, web, zrows, outa, outb,
             src_v, dst_v, prow, ew_s, wv, gsem, ssem, acc):
    c = lax.axis_index("c")
    s = lax.axis_index("s")

    pltpu.sync_copy(zrows, acc.at[pl.ds(s * RPT, RPT)])
    plsc.subcore_barrier()

    def half(p_hbm, w_hbm, out_hbm):
        pltpu.sync_copy(w_hbm, wv)
        w = [[wv[k, pl.ds(jj * 16, 16)] for jj in range(2)] for k in range(3)]

        def sb_body(b, carry):
            rowbase = s * (EPT // SUB) + b * NSUB
            gbase = s * (EPT // 16) + b * GRP
            pltpu.sync_copy(src2d.at[pl.ds(rowbase, NSUB)], src_v)
            pltpu.sync_copy(dst2d.at[pl.ds(rowbase, NSUB)], dst_v)
            pltpu.sync_copy(ewg.at[:, pl.ds(gbase, GRP)], ew_s)
            gh = [
                pltpu.async_copy(p_hbm.at[src_v.at[j]],
                                 prow.at[pl.ds(j * SUB, SUB)], gsem)
                for j in range(NSUB)
            ]
            for hdl in gh:
                hdl.wait()

            def ebody(g, carry2):
                e0 = ew_s[0, g, :]
                e1 = ew_s[1, g, :]
                e2 = ew_s[2, g, :]
                for u in range(16):
                    k = g * 16 + u
                    s0 = e0[u]
                    s1 = e1[u]
                    s2 = e2[u]
                    for jj in range(2):
                        sl = pl.ds(jj * 16, 16)
                        t = prow[k, sl]
                        t = t + s0 * w[0][jj]
                        t = t + s1 * w[1][jj]
                        t = t + s2 * w[2][jj]
                        prow[k, sl] = jnp.maximum(t, 0.0)
                return carry2

            lax.fori_loop(0, GRP, ebody, 0)
            sh = [
                pltpu.async_copy(prow.at[pl.ds(j * SUB, SUB)],
                                 acc.at[dst_v.at[j]], ssem, add=True)
                for j in range(NSUB)
            ]
            for hdl in sh:
                hdl.wait()
            return carry

        lax.fori_loop(0, NSB, sb_body, 0)
        plsc.subcore_barrier()
        pltpu.sync_copy(acc.at[pl.ds(s * RPT, RPT)],
                        out_hbm.at[pl.ds(s * RPT, RPT)])

    @pl.when(c == 0)
    def _():
        half(pa, wea, outa)

    @pl.when(c == 1)
    def _():
        half(pb, web, outb)


_sc_edge = functools.partial(
    pl.kernel,
    out_type=(
        jax.ShapeDtypeStruct((NP, 32), jnp.float32),
        jax.ShapeDtypeStruct((NP, 32), jnp.float32),
    ),
    mesh=plsc.VectorSubcoreMesh(core_axis_name="c", subcore_axis_name="s"),
    compiler_params=pltpu.CompilerParams(use_tc_tiling_on_sc=False),
    scratch_types=[
        pltpu.VMEM((NSUB, SUB), jnp.int32),
        pltpu.VMEM((NSUB, SUB), jnp.int32),
        pltpu.VMEM((SB, 32), jnp.float32),
        pltpu.VMEM((3, GRP, 16), jnp.float32),
        pltpu.VMEM((3, 32), jnp.float32),
        pltpu.SemaphoreType.DMA,
        pltpu.SemaphoreType.DMA,
        pltpu.VMEM_SHARED((NP, 32), jnp.float32),
    ],
)(_sc_body)


def _make_node_body(in_dim, has_next):
    def body(*refs):
        (hp_ref, hna_ref, hnb_ref, gid_ref, w2a_ref, w2b0_ref, w2b1_ref) = refs[:7]
        idx = 7
        if has_next:
            w1na_ref, w1nb_ref, b1na_ref, b1nb_ref = refs[idx:idx + 4]
            idx += 4
        h_ref = refs[idx]
        r_ref = refs[idx + 1]
        if has_next:
            pa_ref, pb_ref = refs[idx + 2:idx + 4]

        ht = (jnp.dot(hp_ref[...], w2a_ref[...], preferred_element_type=jnp.float32)
              + jnp.dot(hna_ref[...], w2b0_ref[...], preferred_element_type=jnp.float32)
              + jnp.dot(hnb_ref[...], w2b1_ref[...], preferred_element_type=jnp.float32))
        hl = jnp.maximum(ht, 0.0)
        ss = jnp.sum(hl * hl, axis=1, keepdims=True)
        nrm = jnp.maximum(jnp.sqrt(ss), 1e-12)
        h = hl / nrm
        h_ref[...] = h
        if has_next:
            pa_ref[...] = (jnp.dot(h, w1na_ref[...],
                                   preferred_element_type=jnp.float32)
                           + b1na_ref[...])
            pb_ref[...] = (jnp.dot(h, w1nb_ref[...],
                                   preferred_element_type=jnp.float32)
                           + b1nb_ref[...])
        gid = gid_ref[...]
        onehot = (gid == lax.broadcasted_iota(jnp.int32, (BLK, G), 1)
                  ).astype(jnp.float32)
        contrib = lax.dot_general(onehot, h, (((0,), (0,)), ((), ())),
                                  preferred_element_type=jnp.float32)

        @pl.when(pl.program_id(0) == 0)
        def _():
            r_ref[...] = jnp.zeros_like(r_ref)

        r_ref[...] += contrib
    return body


def _node_update(h_prev, hna, hnb, gid2d, w2a, w2b0, w2b1, w1n=None, b1n=None):
    in_dim = h_prev.shape[1]
    has_next = w1n is not None
    in_specs = [
        pl.BlockSpec((BLK, in_dim), lambda i: (i, 0)),
        pl.BlockSpec((BLK, 32), lambda i: (i, 0)),
        pl.BlockSpec((BLK, 32), lambda i: (i, 0)),
        pl.BlockSpec((BLK, 1), lambda i: (i, 0)),
        pl.BlockSpec((in_dim, H), lambda i: (0, 0)),
        pl.BlockSpec((32, H), lambda i: (0, 0)),
        pl.BlockSpec((32, H), lambda i: (0, 0)),
    ]
    args = [h_prev, hna, hnb, gid2d, w2a, w2b0, w2b1]
    out_specs = [
        pl.BlockSpec((BLK, H), lambda i: (i, 0)),
        pl.BlockSpec((G, H), lambda i: (0, 0)),
    ]
    out_shape = [
        jax.ShapeDtypeStruct((N, H), jnp.float32),
        jax.ShapeDtypeStruct((G, H), jnp.float32),
    ]
    if has_next:
        in_specs += ([pl.BlockSpec((H, 32), lambda i: (0, 0))] * 2
                     + [pl.BlockSpec((1, 32), lambda i: (0, 0))] * 2)
        args += [w1n[:, :32], w1n[:, 32:], b1n[:, :32], b1n[:, 32:]]
        out_specs += [pl.BlockSpec((BLK, 32), lambda i: (i, 0))] * 2
        out_shape += [jax.ShapeDtypeStruct((N, 32), jnp.float32)] * 2
    return pl.pallas_call(
        _make_node_body(in_dim, has_next),
        grid=(N // BLK,),
        in_specs=in_specs,
        out_specs=out_specs,
        out_shape=out_shape,
    )(*args)


def kernel(gate_type, edge_index, edge_w, graph_id, params):
    layers = params['layers']
    emb = params['emb']

    gate2d = gate_type.reshape(N, 1)
    gid2d = graph_id.reshape(N, 1)
    src2d = edge_index[0].reshape(E // SUB, SUB)
    dst2d = edge_index[1].reshape(E // SUB, SUB)
    ewg = edge_w.T.reshape(3, E // 16, 16)
    zrows = jnp.zeros((RPT, 32), jnp.float32)

    dims = [EMBED, H, H]
    w1_1 = layers[0]['W1']
    b1_1 = layers[0]['b1'].reshape(1, H)
    h0, pa, pb = _embed(gate2d, emb, w1_1[:EMBED, :32], w1_1[:EMBED, 32:],
                        b1_1[:, :32], b1_1[:, 32:])

    h = h0
    readouts = []
    for l in range(3):
        w1e = layers[l]['W1'][dims[l]:, :]
        hna, hnb = _sc_edge(src2d, dst2d, pa, pb, ewg,
                            w1e[:, :32], w1e[:, 32:], zrows)
        w2 = layers[l]['W2']
        in_dim = dims[l]
        w1n = layers[l + 1]['W1'][:H, :] if l < 2 else None
        b1n = layers[l + 1]['b1'].reshape(1, H) if l < 2 else None
        outs = _node_update(h, hna, hnb, gid2d,
                            w2[:in_dim, :], w2[in_dim:in_dim + 32, :],
                            w2[in_dim + 32:, :], w1n, b1n)
        if l < 2:
            h, r_l, pa, pb = outs
        else:
            h, r_l = outs
        readouts.append(r_l)

    return h, jnp.concatenate(readouts, axis=1)


# speedup vs baseline: 4.6812x; 1.1441x over previous
import functools

import jax
import jax.numpy as jnp
from jax import lax
from jax.experimental import pallas as pl
from jax.experimental.pallas import tpu as pltpu
from jax.experimental.pallas import tpu_sc as plsc

N = 50000
E = 800000
G = 50
NGT = 30
EMBED = 16
H = 64

BLK = 1000
SUB = 100
SB = 400
NSUB = SB // SUB
GRP = SB // 16
TILES = 16
EPT = E // TILES
NSB = EPT // SB
NP = 50048
RPT = NP // TILES


def _embed_body(gate_ref, emb_ref, w1ha_ref, w1hb_ref, b1a_ref, b1b_ref,
                h0_ref, pa_ref, pb_ref):
    gate = gate_ref[...]
    onehot = (gate == lax.broadcasted_iota(jnp.int32, (BLK, NGT), 1)
              ).astype(jnp.float32)
    h0 = jnp.dot(onehot, emb_ref[...], preferred_element_type=jnp.float32)
    h0_ref[...] = h0
    pa_ref[...] = (jnp.dot(h0, w1ha_ref[...], preferred_element_type=jnp.float32)
                   + b1a_ref[...])
    pb_ref[...] = (jnp.dot(h0, w1hb_ref[...], preferred_element_type=jnp.float32)
                   + b1b_ref[...])


def _embed(gate2d, emb, w1ha, w1hb, b1a, b1b):
    return pl.pallas_call(
        _embed_body,
        grid=(N // BLK,),
        in_specs=[
            pl.BlockSpec((BLK, 1), lambda i: (i, 0)),
            pl.BlockSpec((NGT, EMBED), lambda i: (0, 0)),
            pl.BlockSpec((EMBED, 32), lambda i: (0, 0)),
            pl.BlockSpec((EMBED, 32), lambda i: (0, 0)),
            pl.BlockSpec((1, 32), lambda i: (0, 0)),
            pl.BlockSpec((1, 32), lambda i: (0, 0)),
        ],
        out_specs=[
            pl.BlockSpec((BLK, EMBED), lambda i: (i, 0)),
            pl.BlockSpec((BLK, 32), lambda i: (i, 0)),
            pl.BlockSpec((BLK, 32), lambda i: (i, 0)),
        ],
        out_shape=[
            jax.ShapeDtypeStruct((N, EMBED), jnp.float32),
            jax.ShapeDtypeStruct((N, 32), jnp.float32),
            jax.ShapeDtypeStruct((N, 32), jnp.float32),
        ],
    )(gate2d, emb, w1ha, w1hb, b1a, b1b)


def _sc_body(src2d, dst2d, pa, pb, ewg, wea, web, zrows, outa, outb,
             src_vA, dst_vA, prowA, ew_sA,
             src_vB, dst_vB, prowB, ew_sB,
             wv, gsemA, gsemB, ssemA, ssemB, acc):
    c = lax.axis_index("c")
    s = lax.axis_index("s")

    pltpu.sync_copy(zrows, acc.at[pl.ds(s * RPT, RPT)])
    plsc.subcore_barrier()

    bufs = ((src_vA, dst_vA, prowA, ew_sA, gsemA, ssemA),
            (src_vB, dst_vB, prowB, ew_sB, gsemB, ssemB))

    def half(p_hbm, w_hbm, out_hbm):
        pltpu.sync_copy(w_hbm, wv)
        w = [[wv[k, pl.ds(jj * 16, 16)] for jj in range(2)] for k in range(3)]

        def fetch_streams(b, buf):
            src_v, dst_v, _, ew_s, _, _ = buf
            rowbase = s * (EPT // SUB) + b * NSUB
            gbase = s * (EPT // 16) + b * GRP
            pltpu.sync_copy(src2d.at[pl.ds(rowbase, NSUB)], src_v)
            pltpu.sync_copy(dst2d.at[pl.ds(rowbase, NSUB)], dst_v)
            pltpu.sync_copy(ewg.at[:, pl.ds(gbase, GRP)], ew_s)

        def start_gather(buf):
            src_v, _, prow, _, gsem, _ = buf
            for j in range(NSUB):
                pltpu.async_copy(p_hbm.at[src_v.at[j]],
                                 prow.at[pl.ds(j * SUB, SUB)], gsem)

        def drain(buf, sem):
            src_v, dst_v, prow, _, gsem, ssem = buf
            for j in range(NSUB):
                sl = pl.ds(j * SUB, SUB)
                if sem == 'g':
                    pltpu.make_async_copy(p_hbm.at[src_v.at[j]],
                                          prow.at[sl], gsem).wait()
                else:
                    pltpu.make_async_copy(prow.at[sl],
                                          acc.at[dst_v.at[j]], ssem).wait()

        def compute_scatter(buf):
            _, dst_v, prow, ew_s, _, ssem = buf

            def ebody(g, carry2):
                e0 = ew_s[0, g, :]
                e1 = ew_s[1, g, :]
                e2 = ew_s[2, g, :]
                for u in range(16):
                    k = g * 16 + u
                    s0 = e0[u]
                    s1 = e1[u]
                    s2 = e2[u]
                    for jj in range(2):
                        sl = pl.ds(jj * 16, 16)
                        t = prow[k, sl]
                        t = t + s0 * w[0][jj]
                        t = t + s1 * w[1][jj]
                        t = t + s2 * w[2][jj]
                        prow[k, sl] = jnp.maximum(t, 0.0)
                return carry2

            lax.fori_loop(0, GRP, ebody, 0)
            for j in range(NSUB):
                pltpu.async_copy(prow.at[pl.ds(j * SUB, SUB)],
                                 acc.at[dst_v.at[j]], ssem, add=True)

        fetch_streams(0, bufs[0])
        start_gather(bufs[0])

        def sb_body(i, carry):
            def step(cur, nxt):
                @pl.when(i >= 1)
                def _():
                    drain(nxt, 's')

                fetch_streams(i + 1, nxt)
                start_gather(nxt)
                drain(cur, 'g')
                compute_scatter(cur)

            @pl.when(lax.rem(i, 2) == 0)
            def _():
                step(bufs[0], bufs[1])

            @pl.when(lax.rem(i, 2) == 1)
            def _():
                step(bufs[1], bufs[0])

            return carry

        lax.fori_loop(0, NSB - 1, sb_body, 0)
        last = bufs[(NSB - 1) % 2]
        prev = bufs[NSB % 2]
        drain(prev, 's')
        drain(last, 'g')
        compute_scatter(last)
        drain(last, 's')
        plsc.subcore_barrier()
        pltpu.sync_copy(acc.at[pl.ds(s * RPT, RPT)],
                        out_hbm.at[pl.ds(s * RPT, RPT)])

    @pl.when(c == 0)
    def _():
        half(pa, wea, outa)

    @pl.when(c == 1)
    def _():
        half(pb, web, outb)


_sc_edge = functools.partial(
    pl.kernel,
    out_type=(
        jax.ShapeDtypeStruct((NP, 32), jnp.float32),
        jax.ShapeDtypeStruct((NP, 32), jnp.float32),
    ),
    mesh=plsc.VectorSubcoreMesh(core_axis_name="c", subcore_axis_name="s"),
    compiler_params=pltpu.CompilerParams(use_tc_tiling_on_sc=False),
    scratch_types=[
        pltpu.VMEM((NSUB, SUB), jnp.int32),
        pltpu.VMEM((NSUB, SUB), jnp.int32),
        pltpu.VMEM((SB, 32), jnp.float32),
        pltpu.VMEM((3, GRP, 16), jnp.float32),
        pltpu.VMEM((NSUB, SUB), jnp.int32),
        pltpu.VMEM((NSUB, SUB), jnp.int32),
        pltpu.VMEM((SB, 32), jnp.float32),
        pltpu.VMEM((3, GRP, 16), jnp.float32),
        pltpu.VMEM((3, 32), jnp.float32),
        pltpu.SemaphoreType.DMA,
        pltpu.SemaphoreType.DMA,
        pltpu.SemaphoreType.DMA,
        pltpu.SemaphoreType.DMA,
        pltpu.VMEM_SHARED((NP, 32), jnp.float32),
    ],
)(_sc_body)


def _make_node_body(in_dim, has_next):
    def body(*refs):
        (hp_ref, hna_ref, hnb_ref, gid_ref, w2a_ref, w2b0_ref, w2b1_ref) = refs[:7]
        idx = 7
        if has_next:
            w1na_ref, w1nb_ref, b1na_ref, b1nb_ref = refs[idx:idx + 4]
            idx += 4
        h_ref = refs[idx]
        r_ref = refs[idx + 1]
        if has_next:
            pa_ref, pb_ref = refs[idx + 2:idx + 4]

        ht = (jnp.dot(hp_ref[...], w2a_ref[...], preferred_element_type=jnp.float32)
              + jnp.dot(hna_ref[...], w2b0_ref[...], preferred_element_type=jnp.float32)
              + jnp.dot(hnb_ref[...], w2b1_ref[...], preferred_element_type=jnp.float32))
        hl = jnp.maximum(ht, 0.0)
        ss = jnp.sum(hl * hl, axis=1, keepdims=True)
        nrm = jnp.maximum(jnp.sqrt(ss), 1e-12)
        h = hl / nrm
        h_ref[...] = h
        if has_next:
            pa_ref[...] = (jnp.dot(h, w1na_ref[...],
                                   preferred_element_type=jnp.float32)
                           + b1na_ref[...])
            pb_ref[...] = (jnp.dot(h, w1nb_ref[...],
                                   preferred_element_type=jnp.float32)
                           + b1nb_ref[...])
        gid = gid_ref[...]
        onehot = (gid == lax.broadcasted_iota(jnp.int32, (BLK, G), 1)
                  ).astype(jnp.float32)
        contrib = lax.dot_general(onehot, h, (((0,), (0,)), ((), ())),
                                  preferred_element_type=jnp.float32)

        @pl.when(pl.program_id(0) == 0)
        def _():
            r_ref[...] = jnp.zeros_like(r_ref)

        r_ref[...] += contrib
    return body


def _node_update(h_prev, hna, hnb, gid2d, w2a, w2b0, w2b1, w1n=None, b1n=None):
    in_dim = h_prev.shape[1]
    has_next = w1n is not None
    in_specs = [
        pl.BlockSpec((BLK, in_dim), lambda i: (i, 0)),
        pl.BlockSpec((BLK, 32), lambda i: (i, 0)),
        pl.BlockSpec((BLK, 32), lambda i: (i, 0)),
        pl.BlockSpec((BLK, 1), lambda i: (i, 0)),
        pl.BlockSpec((in_dim, H), lambda i: (0, 0)),
        pl.BlockSpec((32, H), lambda i: (0, 0)),
        pl.BlockSpec((32, H), lambda i: (0, 0)),
    ]
    args = [h_prev, hna, hnb, gid2d, w2a, w2b0, w2b1]
    out_specs = [
        pl.BlockSpec((BLK, H), lambda i: (i, 0)),
        pl.BlockSpec((G, H), lambda i: (0, 0)),
    ]
    out_shape = [
        jax.ShapeDtypeStruct((N, H), jnp.float32),
        jax.ShapeDtypeStruct((G, H), jnp.float32),
    ]
    if has_next:
        in_specs += ([pl.BlockSpec((H, 32), lambda i: (0, 0))] * 2
                     + [pl.BlockSpec((1, 32), lambda i: (0, 0))] * 2)
        args += [w1n[:, :32], w1n[:, 32:], b1n[:, :32], b1n[:, 32:]]
        out_specs += [pl.BlockSpec((BLK, 32), lambda i: (i, 0))] * 2
        out_shape += [jax.ShapeDtypeStruct((N, 32), jnp.float32)] * 2
    return pl.pallas_call(
        _make_node_body(in_dim, has_next),
        grid=(N // BLK,),
        in_specs=in_specs,
        out_specs=out_specs,
        out_shape=out_shape,
    )(*args)


def kernel(gate_type, edge_index, edge_w, graph_id, params):
    layers = params['layers']
    emb = params['emb']

    gate2d = gate_type.reshape(N, 1)
    gid2d = graph_id.reshape(N, 1)
    src2d = edge_index[0].reshape(E // SUB, SUB)
    dst2d = edge_index[1].reshape(E // SUB, SUB)
    ewg = edge_w.T.reshape(3, E // 16, 16)
    zrows = jnp.zeros((RPT, 32), jnp.float32)

    dims = [EMBED, H, H]
    w1_1 = layers[0]['W1']
    b1_1 = layers[0]['b1'].reshape(1, H)
    h0, pa, pb = _embed(gate2d, emb, w1_1[:EMBED, :32], w1_1[:EMBED, 32:],
                        b1_1[:, :32], b1_1[:, 32:])

    h = h0
    readouts = []
    for l in range(3):
        w1e = layers[l]['W1'][dims[l]:, :]
        hna, hnb = _sc_edge(src2d, dst2d, pa, pb, ewg,
                            w1e[:, :32], w1e[:, 32:], zrows)
        w2 = layers[l]['W2']
        in_dim = dims[l]
        w1n = layers[l + 1]['W1'][:H, :] if l < 2 else None
        b1n = layers[l + 1]['b1'].reshape(1, H) if l < 2 else None
        outs = _node_update(h, hna, hnb, gid2d,
                            w2[:in_dim, :], w2[in_dim:in_dim + 32, :],
                            w2[in_dim + 32:, :], w1n, b1n)
        if l < 2:
            h, r_l, pa, pb = outs
        else:
            h, r_l = outs
        readouts.append(r_l)

    return h, jnp.concatenate(readouts, axis=1)


# speedup vs baseline: 5.3859x; 1.1505x over previous
import functools

import jax
import jax.numpy as jnp
from jax import lax
from jax.experimental import pallas as pl
from jax.experimental.pallas import tpu as pltpu
from jax.experimental.pallas import tpu_sc as plsc

N = 50000
E = 800000
G = 50
NGT = 30
EMBED = 16
H = 64

BLK = 1000
SUB = 100
SB = 400
NSUB = SB // SUB
GRP = SB // 16
TILES = 16
EPT = E // TILES
NSB = EPT // SB
NP = 50048
RPT = NP // TILES


def _embed_body(gate_ref, emb_ref, w1ha_ref, w1hb_ref, b1a_ref, b1b_ref,
                h0_ref, pa_ref, pb_ref):
    gate = gate_ref[...]
    onehot = (gate == lax.broadcasted_iota(jnp.int32, (BLK, NGT), 1)
              ).astype(jnp.float32)
    h0 = jnp.dot(onehot, emb_ref[...], preferred_element_type=jnp.float32)
    h0_ref[...] = h0
    pa_ref[...] = (jnp.dot(h0, w1ha_ref[...], preferred_element_type=jnp.float32)
                   + b1a_ref[...])
    pb_ref[...] = (jnp.dot(h0, w1hb_ref[...], preferred_element_type=jnp.float32)
                   + b1b_ref[...])


def _embed(gate2d, emb, w1ha, w1hb, b1a, b1b):
    return pl.pallas_call(
        _embed_body,
        grid=(N // BLK,),
        in_specs=[
            pl.BlockSpec((BLK, 1), lambda i: (i, 0)),
            pl.BlockSpec((NGT, EMBED), lambda i: (0, 0)),
            pl.BlockSpec((EMBED, 32), lambda i: (0, 0)),
            pl.BlockSpec((EMBED, 32), lambda i: (0, 0)),
            pl.BlockSpec((1, 32), lambda i: (0, 0)),
            pl.BlockSpec((1, 32), lambda i: (0, 0)),
        ],
        out_specs=[
            pl.BlockSpec((BLK, EMBED), lambda i: (i, 0)),
            pl.BlockSpec((BLK, 32), lambda i: (i, 0)),
            pl.BlockSpec((BLK, 32), lambda i: (i, 0)),
        ],
        out_shape=[
            jax.ShapeDtypeStruct((N, EMBED), jnp.float32),
            jax.ShapeDtypeStruct((N, 32), jnp.float32),
            jax.ShapeDtypeStruct((N, 32), jnp.float32),
        ],
    )(gate2d, emb, w1ha, w1hb, b1a, b1b)


def _sc_body(src2d, dst2d, pa, pb, ewg, wea, web, zrows, outa, outb,
             src_vA, dst_vA, prowA, ew_sA,
             src_vB, dst_vB, prowB, ew_sB,
             wv, gsemA, gsemB, ssemA, ssemB, fsemA, fsemB, acc):
    c = lax.axis_index("c")
    s = lax.axis_index("s")

    pltpu.sync_copy(zrows, acc.at[pl.ds(s * RPT, RPT)])
    plsc.subcore_barrier()

    bufs = ((src_vA, dst_vA, prowA, ew_sA, gsemA, ssemA, fsemA),
            (src_vB, dst_vB, prowB, ew_sB, gsemB, ssemB, fsemB))

    def half(p_hbm, w_hbm, out_hbm):
        pltpu.sync_copy(w_hbm, wv)
        w = [[wv[k, pl.ds(jj * 16, 16)] for jj in range(2)] for k in range(3)]

        def srcew_copies(b, buf):
            src_v, _, _, ew_s, _, _, fsem = buf
            rowbase = s * (EPT // SUB) + b * NSUB
            gbase = s * (EPT // 16) + b * GRP
            return ((src2d.at[pl.ds(rowbase, NSUB)], src_v, fsem),
                    (ewg.at[:, pl.ds(gbase, GRP)], ew_s, fsem))

        def start_srcew(b, buf):
            for args in srcew_copies(b, buf):
                pltpu.async_copy(*args)

        def drain_srcew(b, buf):
            for args in srcew_copies(b, buf):
                pltpu.make_async_copy(*args).wait()

        def fetch_dst(b, buf):
            dst_v = buf[1]
            rowbase = s * (EPT // SUB) + b * NSUB
            pltpu.sync_copy(dst2d.at[pl.ds(rowbase, NSUB)], dst_v)

        def start_gather(buf):
            src_v, _, prow, _, _, _, _ = buf
            gsem = buf[4]
            for j in range(NSUB):
                pltpu.async_copy(p_hbm.at[src_v.at[j]],
                                 prow.at[pl.ds(j * SUB, SUB)], gsem)

        def drain(buf, sem):
            src_v, dst_v, prow, _, gsem, ssem, _ = buf
            for j in range(NSUB):
                sl = pl.ds(j * SUB, SUB)
                if sem == 'g':
                    pltpu.make_async_copy(p_hbm.at[src_v.at[j]],
                                          prow.at[sl], gsem).wait()
                else:
                    pltpu.make_async_copy(prow.at[sl],
                                          acc.at[dst_v.at[j]], ssem).wait()

        def compute_scatter(buf):
            _, dst_v, prow, ew_s, _, ssem, _ = buf

            def ebody(g, carry2):
                e0 = ew_s[0, g, :]
                e1 = ew_s[1, g, :]
                e2 = ew_s[2, g, :]
                for u in range(16):
                    k = g * 16 + u
                    s0 = e0[u]
                    s1 = e1[u]
                    s2 = e2[u]
                    for jj in range(2):
                        sl = pl.ds(jj * 16, 16)
                        t = prow[k, sl]
                        t = t + s0 * w[0][jj]
                        t = t + s1 * w[1][jj]
                        t = t + s2 * w[2][jj]
                        prow[k, sl] = jnp.maximum(t, 0.0)
                return carry2

            lax.fori_loop(0, GRP, ebody, 0)
            for j in range(NSUB):
                pltpu.async_copy(prow.at[pl.ds(j * SUB, SUB)],
                                 acc.at[dst_v.at[j]], ssem, add=True)

        start_srcew(0, bufs[0])
        drain_srcew(0, bufs[0])
        fetch_dst(0, bufs[0])
        start_gather(bufs[0])
        start_srcew(1, bufs[1])

        def sb_body(i, carry):
            def step(cur, nxt):
                drain_srcew(i + 1, nxt)

                @pl.when(i >= 1)
                def _():
                    drain(nxt, 's')

                fetch_dst(i + 1, nxt)
                start_gather(nxt)
                drain(cur, 'g')
                compute_scatter(cur)

                @pl.when(i < NSB - 2)
                def _():
                    start_srcew(i + 2, cur)

            @pl.when(lax.rem(i, 2) == 0)
            def _():
                step(bufs[0], bufs[1])

            @pl.when(lax.rem(i, 2) == 1)
            def _():
                step(bufs[1], bufs[0])

            return carry

        lax.fori_loop(0, NSB - 1, sb_body, 0)
        last = bufs[(NSB - 1) % 2]
        prev = bufs[NSB % 2]
        drain(prev, 's')
        drain(last, 'g')
        compute_scatter(last)
        drain(last, 's')
        plsc.subcore_barrier()
        pltpu.sync_copy(acc.at[pl.ds(s * RPT, RPT)],
                        out_hbm.at[pl.ds(s * RPT, RPT)])

    @pl.when(c == 0)
    def _():
        half(pa, wea, outa)

    @pl.when(c == 1)
    def _():
        half(pb, web, outb)


_sc_edge = functools.partial(
    pl.kernel,
    out_type=(
        jax.ShapeDtypeStruct((NP, 32), jnp.float32),
        jax.ShapeDtypeStruct((NP, 32), jnp.float32),
    ),
    mesh=plsc.VectorSubcoreMesh(core_axis_name="c", subcore_axis_name="s"),
    compiler_params=pltpu.CompilerParams(use_tc_tiling_on_sc=False),
    scratch_types=[
        pltpu.VMEM((NSUB, SUB), jnp.int32),
        pltpu.VMEM((NSUB, SUB), jnp.int32),
        pltpu.VMEM((SB, 32), jnp.float32),
        pltpu.VMEM((3, GRP, 16), jnp.float32),
        pltpu.VMEM((NSUB, SUB), jnp.int32),
        pltpu.VMEM((NSUB, SUB), jnp.int32),
        pltpu.VMEM((SB, 32), jnp.float32),
        pltpu.VMEM((3, GRP, 16), jnp.float32),
        pltpu.VMEM((3, 32), jnp.float32),
        pltpu.SemaphoreType.DMA,
        pltpu.SemaphoreType.DMA,
        pltpu.SemaphoreType.DMA,
        pltpu.SemaphoreType.DMA,
        pltpu.SemaphoreType.DMA,
        pltpu.SemaphoreType.DMA,
        pltpu.VMEM_SHARED((NP, 32), jnp.float32),
    ],
)(_sc_body)


def _make_node_body(in_dim, has_next):
    def body(*refs):
        (hp_ref, hna_ref, hnb_ref, gid_ref, w2a_ref, w2b0_ref, w2b1_ref) = refs[:7]
        idx = 7
        if has_next:
            w1na_ref, w1nb_ref, b1na_ref, b1nb_ref = refs[idx:idx + 4]
            idx += 4
        h_ref = refs[idx]
        r_ref = refs[idx + 1]
        if has_next:
            pa_ref, pb_ref = refs[idx + 2:idx + 4]

        ht = (jnp.dot(hp_ref[...], w2a_ref[...], preferred_element_type=jnp.float32)
              + jnp.dot(hna_ref[...], w2b0_ref[...], preferred_element_type=jnp.float32)
              + jnp.dot(hnb_ref[...], w2b1_ref[...], preferred_element_type=jnp.float32))
        hl = jnp.maximum(ht, 0.0)
        ss = jnp.sum(hl * hl, axis=1, keepdims=True)
        nrm = jnp.maximum(jnp.sqrt(ss), 1e-12)
        h = hl / nrm
        h_ref[...] = h
        if has_next:
            pa_ref[...] = (jnp.dot(h, w1na_ref[...],
                                   preferred_element_type=jnp.float32)
                           + b1na_ref[...])
            pb_ref[...] = (jnp.dot(h, w1nb_ref[...],
                                   preferred_element_type=jnp.float32)
                           + b1nb_ref[...])
        gid = gid_ref[...]
        onehot = (gid == lax.broadcasted_iota(jnp.int32, (BLK, G), 1)
                  ).astype(jnp.float32)
        contrib = lax.dot_general(onehot, h, (((0,), (0,)), ((), ())),
                                  preferred_element_type=jnp.float32)

        @pl.when(pl.program_id(0) == 0)
        def _():
            r_ref[...] = jnp.zeros_like(r_ref)

        r_ref[...] += contrib
    return body


def _node_update(h_prev, hna, hnb, gid2d, w2a, w2b0, w2b1, w1n=None, b1n=None):
    in_dim = h_prev.shape[1]
    has_next = w1n is not None
    in_specs = [
        pl.BlockSpec((BLK, in_dim), lambda i: (i, 0)),
        pl.BlockSpec((BLK, 32), lambda i: (i, 0)),
        pl.BlockSpec((BLK, 32), lambda i: (i, 0)),
        pl.BlockSpec((BLK, 1), lambda i: (i, 0)),
        pl.BlockSpec((in_dim, H), lambda i: (0, 0)),
        pl.BlockSpec((32, H), lambda i: (0, 0)),
        pl.BlockSpec((32, H), lambda i: (0, 0)),
    ]
    args = [h_prev, hna, hnb, gid2d, w2a, w2b0, w2b1]
    out_specs = [
        pl.BlockSpec((BLK, H), lambda i: (i, 0)),
        pl.BlockSpec((G, H), lambda i: (0, 0)),
    ]
    out_shape = [
        jax.ShapeDtypeStruct((N, H), jnp.float32),
        jax.ShapeDtypeStruct((G, H), jnp.float32),
    ]
    if has_next:
        in_specs += ([pl.BlockSpec((H, 32), lambda i: (0, 0))] * 2
                     + [pl.BlockSpec((1, 32), lambda i: (0, 0))] * 2)
        args += [w1n[:, :32], w1n[:, 32:], b1n[:, :32], b1n[:, 32:]]
        out_specs += [pl.BlockSpec((BLK, 32), lambda i: (i, 0))] * 2
        out_shape += [jax.ShapeDtypeStruct((N, 32), jnp.float32)] * 2
    return pl.pallas_call(
        _make_node_body(in_dim, has_next),
        grid=(N // BLK,),
        in_specs=in_specs,
        out_specs=out_specs,
        out_shape=out_shape,
    )(*args)


def kernel(gate_type, edge_index, edge_w, graph_id, params):
    layers = params['layers']
    emb = params['emb']

    gate2d = gate_type.reshape(N, 1)
    gid2d = graph_id.reshape(N, 1)
    src2d = edge_index[0].reshape(E // SUB, SUB)
    dst2d = edge_index[1].reshape(E // SUB, SUB)
    ewg = edge_w.T.reshape(3, E // 16, 16)
    zrows = jnp.zeros((RPT, 32), jnp.float32)

    dims = [EMBED, H, H]
    w1_1 = layers[0]['W1']
    b1_1 = layers[0]['b1'].reshape(1, H)
    h0, pa, pb = _embed(gate2d, emb, w1_1[:EMBED, :32], w1_1[:EMBED, 32:],
                        b1_1[:, :32], b1_1[:, 32:])

    h = h0
    readouts = []
    for l in range(3):
        w1e = layers[l]['W1'][dims[l]:, :]
        hna, hnb = _sc_edge(src2d, dst2d, pa, pb, ewg,
                            w1e[:, :32], w1e[:, 32:], zrows)
        w2 = layers[l]['W2']
        in_dim = dims[l]
        w1n = layers[l + 1]['W1'][:H, :] if l < 2 else None
        b1n = layers[l + 1]['b1'].reshape(1, H) if l < 2 else None
        outs = _node_update(h, hna, hnb, gid2d,
                            w2[:in_dim, :], w2[in_dim:in_dim + 32, :],
                            w2[in_dim + 32:, :], w1n, b1n)
        if l < 2:
            h, r_l, pa, pb = outs
        else:
            h, r_l = outs
        readouts.append(r_l)

    return h, jnp.concatenate(readouts, axis=1)


# speedup vs baseline: 5.8565x; 1.0874x over previous
import functools

import jax
import jax.numpy as jnp
from jax import lax
from jax.experimental import pallas as pl
from jax.experimental.pallas import tpu as pltpu
from jax.experimental.pallas import tpu_sc as plsc

N = 50000
E = 800000
G = 50
NGT = 30
EMBED = 16
H = 64

BLK = 1000
SUB = 100
SB = 400
NSUB = SB // SUB
GRP = SB // 16
TILES = 16
EPT = E // TILES
NSB = EPT // SB
NP = 50048
RPT = NP // TILES


def _embed_body(gate_ref, emb_ref, w1ha_ref, w1hb_ref, b1a_ref, b1b_ref,
                h0_ref, pa_ref, pb_ref):
    gate = gate_ref[...]
    onehot = (gate == lax.broadcasted_iota(jnp.int32, (BLK, NGT), 1)
              ).astype(jnp.float32)
    h0 = jnp.dot(onehot, emb_ref[...], preferred_element_type=jnp.float32)
    h0_ref[...] = h0
    pa_ref[...] = (jnp.dot(h0, w1ha_ref[...], preferred_element_type=jnp.float32)
                   + b1a_ref[...])
    pb_ref[...] = (jnp.dot(h0, w1hb_ref[...], preferred_element_type=jnp.float32)
                   + b1b_ref[...])


def _embed(gate2d, emb, w1ha, w1hb, b1a, b1b):
    return pl.pallas_call(
        _embed_body,
        grid=(N // BLK,),
        in_specs=[
            pl.BlockSpec((BLK, 1), lambda i: (i, 0)),
            pl.BlockSpec((NGT, EMBED), lambda i: (0, 0)),
            pl.BlockSpec((EMBED, 32), lambda i: (0, 0)),
            pl.BlockSpec((EMBED, 32), lambda i: (0, 0)),
            pl.BlockSpec((1, 32), lambda i: (0, 0)),
            pl.BlockSpec((1, 32), lambda i: (0, 0)),
        ],
        out_specs=[
            pl.BlockSpec((BLK, EMBED), lambda i: (i, 0)),
            pl.BlockSpec((BLK, 32), lambda i: (i, 0)),
            pl.BlockSpec((BLK, 32), lambda i: (i, 0)),
        ],
        out_shape=[
            jax.ShapeDtypeStruct((N, EMBED), jnp.float32),
            jax.ShapeDtypeStruct((N, 32), jnp.float32),
            jax.ShapeDtypeStruct((N, 32), jnp.float32),
        ],
    )(gate2d, emb, w1ha, w1hb, b1a, b1b)


def _sc_body(src2d, dst2d, pa, pb, ewg, wea, web, zrows, outa, outb,
             src_vA, dst_vA, prowA, ew_sA,
             src_vB, dst_vB, prowB, ew_sB,
             wv, gsemA, gsemB, ssemA, ssemB, fsemA, fsemB, dsemA, dsemB, acc):
    c = lax.axis_index("c")
    s = lax.axis_index("s")

    pltpu.sync_copy(zrows, acc.at[pl.ds(s * RPT, RPT)])
    plsc.subcore_barrier()

    bufs = ((src_vA, dst_vA, prowA, ew_sA, gsemA, ssemA, fsemA, dsemA),
            (src_vB, dst_vB, prowB, ew_sB, gsemB, ssemB, fsemB, dsemB))

    def half(p_hbm, w_hbm, out_hbm):
        pltpu.sync_copy(w_hbm, wv)
        w = [[wv[k, pl.ds(jj * 16, 16)] for jj in range(2)] for k in range(3)]

        def srcew_copies(b, buf):
            src_v, ew_s, fsem = buf[0], buf[3], buf[6]
            rowbase = s * (EPT // SUB) + b * NSUB
            gbase = s * (EPT // 16) + b * GRP
            return ((src2d.at[pl.ds(rowbase, NSUB)], src_v, fsem),
                    (ewg.at[:, pl.ds(gbase, GRP)], ew_s, fsem))

        def start_srcew(b, buf):
            for args in srcew_copies(b, buf):
                pltpu.async_copy(*args)

        def drain_srcew(b, buf):
            for args in srcew_copies(b, buf):
                pltpu.make_async_copy(*args).wait()

        def dst_copy(b, buf):
            dst_v, dsem = buf[1], buf[7]
            rowbase = s * (EPT // SUB) + b * NSUB
            return (dst2d.at[pl.ds(rowbase, NSUB)], dst_v, dsem)

        def start_gather(buf):
            src_v, prow = buf[0], buf[2]
            gsem = buf[4]
            for j in range(NSUB):
                pltpu.async_copy(p_hbm.at[src_v.at[j]],
                                 prow.at[pl.ds(j * SUB, SUB)], gsem)

        def drain(buf, sem):
            src_v, dst_v, prow = buf[0], buf[1], buf[2]
            gsem, ssem = buf[4], buf[5]
            for j in range(NSUB):
                sl = pl.ds(j * SUB, SUB)
                if sem == 'g':
                    pltpu.make_async_copy(p_hbm.at[src_v.at[j]],
                                          prow.at[sl], gsem).wait()
                else:
                    pltpu.make_async_copy(prow.at[sl],
                                          acc.at[dst_v.at[j]], ssem).wait()

        def compute_scatter(buf):
            dst_v, prow, ew_s, ssem = buf[1], buf[2], buf[3], buf[5]

            def ebody(g, carry2):
                e0 = ew_s[0, g, :]
                e1 = ew_s[1, g, :]
                e2 = ew_s[2, g, :]
                for u in range(16):
                    k = g * 16 + u
                    s0 = e0[u]
                    s1 = e1[u]
                    s2 = e2[u]
                    for jj in range(2):
                        sl = pl.ds(jj * 16, 16)
                        t = prow[k, sl]
                        t = t + s0 * w[0][jj]
                        t = t + s1 * w[1][jj]
                        t = t + s2 * w[2][jj]
                        prow[k, sl] = jnp.maximum(t, 0.0)
                return carry2

            lax.fori_loop(0, GRP, ebody, 0)
            for j in range(NSUB):
                pltpu.async_copy(prow.at[pl.ds(j * SUB, SUB)],
                                 acc.at[dst_v.at[j]], ssem, add=True)

        start_srcew(0, bufs[0])
        drain_srcew(0, bufs[0])
        pltpu.async_copy(*dst_copy(0, bufs[0]))
        start_gather(bufs[0])
        start_srcew(1, bufs[1])

        def sb_body(i, carry):
            def step(cur, nxt):
                drain_srcew(i + 1, nxt)

                @pl.when(i >= 1)
                def _():
                    drain(nxt, 's')

                pltpu.async_copy(*dst_copy(i + 1, nxt))
                start_gather(nxt)
                drain(cur, 'g')
                pltpu.make_async_copy(*dst_copy(i, cur)).wait()
                compute_scatter(cur)

                @pl.when(i < NSB - 2)
                def _():
                    start_srcew(i + 2, cur)

            @pl.when(lax.rem(i, 2) == 0)
            def _():
                step(bufs[0], bufs[1])

            @pl.when(lax.rem(i, 2) == 1)
            def _():
                step(bufs[1], bufs[0])

            return carry

        lax.fori_loop(0, NSB - 1, sb_body, 0)
        last = bufs[(NSB - 1) % 2]
        prev = bufs[NSB % 2]
        drain(prev, 's')
        drain(last, 'g')
        pltpu.make_async_copy(*dst_copy(NSB - 1, last)).wait()
        compute_scatter(last)
        drain(last, 's')
        plsc.subcore_barrier()
        pltpu.sync_copy(acc.at[pl.ds(s * RPT, RPT)],
                        out_hbm.at[pl.ds(s * RPT, RPT)])

    @pl.when(c == 0)
    def _():
        half(pa, wea, outa)

    @pl.when(c == 1)
    def _():
        half(pb, web, outb)


_sc_edge = functools.partial(
    pl.kernel,
    out_type=(
        jax.ShapeDtypeStruct((NP, 32), jnp.float32),
        jax.ShapeDtypeStruct((NP, 32), jnp.float32),
    ),
    mesh=plsc.VectorSubcoreMesh(core_axis_name="c", subcore_axis_name="s"),
    compiler_params=pltpu.CompilerParams(use_tc_tiling_on_sc=False),
    scratch_types=[
        pltpu.VMEM((NSUB, SUB), jnp.int32),
        pltpu.VMEM((NSUB, SUB), jnp.int32),
        pltpu.VMEM((SB, 32), jnp.float32),
        pltpu.VMEM((3, GRP, 16), jnp.float32),
        pltpu.VMEM((NSUB, SUB), jnp.int32),
        pltpu.VMEM((NSUB, SUB), jnp.int32),
        pltpu.VMEM((SB, 32), jnp.float32),
        pltpu.VMEM((3, GRP, 16), jnp.float32),
        pltpu.VMEM((3, 32), jnp.float32),
        pltpu.SemaphoreType.DMA,
        pltpu.SemaphoreType.DMA,
        pltpu.SemaphoreType.DMA,
        pltpu.SemaphoreType.DMA,
        pltpu.SemaphoreType.DMA,
        pltpu.SemaphoreType.DMA,
        pltpu.SemaphoreType.DMA,
        pltpu.SemaphoreType.DMA,
        pltpu.VMEM_SHARED((NP, 32), jnp.float32),
    ],
)(_sc_body)


def _make_node_body(in_dim, has_next):
    def body(*refs):
        (hp_ref, hna_ref, hnb_ref, gid_ref, w2a_ref, w2b0_ref, w2b1_ref) = refs[:7]
        idx = 7
        if has_next:
            w1na_ref, w1nb_ref, b1na_ref, b1nb_ref = refs[idx:idx + 4]
            idx += 4
        h_ref = refs[idx]
        r_ref = refs[idx + 1]
        if has_next:
            pa_ref, pb_ref = refs[idx + 2:idx + 4]

        ht = (jnp.dot(hp_ref[...], w2a_ref[...], preferred_element_type=jnp.float32)
              + jnp.dot(hna_ref[...], w2b0_ref[...], preferred_element_type=jnp.float32)
              + jnp.dot(hnb_ref[...], w2b1_ref[...], preferred_element_type=jnp.float32))
        hl = jnp.maximum(ht, 0.0)
        ss = jnp.sum(hl * hl, axis=1, keepdims=True)
        nrm = jnp.maximum(jnp.sqrt(ss), 1e-12)
        h = hl / nrm
        h_ref[...] = h
        if has_next:
            pa_ref[...] = (jnp.dot(h, w1na_ref[...],
                                   preferred_element_type=jnp.float32)
                           + b1na_ref[...])
            pb_ref[...] = (jnp.dot(h, w1nb_ref[...],
                                   preferred_element_type=jnp.float32)
                           + b1nb_ref[...])
        gid = gid_ref[...]
        onehot = (gid == lax.broadcasted_iota(jnp.int32, (BLK, G), 1)
                  ).astype(jnp.float32)
        contrib = lax.dot_general(onehot, h, (((0,), (0,)), ((), ())),
                                  preferred_element_type=jnp.float32)

        @pl.when(pl.program_id(0) == 0)
        def _():
            r_ref[...] = jnp.zeros_like(r_ref)

        r_ref[...] += contrib
    return body


def _node_update(h_prev, hna, hnb, gid2d, w2a, w2b0, w2b1, w1n=None, b1n=None):
    in_dim = h_prev.shape[1]
    has_next = w1n is not None
    in_specs = [
        pl.BlockSpec((BLK, in_dim), lambda i: (i, 0)),
        pl.BlockSpec((BLK, 32), lambda i: (i, 0)),
        pl.BlockSpec((BLK, 32), lambda i: (i, 0)),
        pl.BlockSpec((BLK, 1), lambda i: (i, 0)),
        pl.BlockSpec((in_dim, H), lambda i: (0, 0)),
        pl.BlockSpec((32, H), lambda i: (0, 0)),
        pl.BlockSpec((32, H), lambda i: (0, 0)),
    ]
    args = [h_prev, hna, hnb, gid2d, w2a, w2b0, w2b1]
    out_specs = [
        pl.BlockSpec((BLK, H), lambda i: (i, 0)),
        pl.BlockSpec((G, H), lambda i: (0, 0)),
    ]
    out_shape = [
        jax.ShapeDtypeStruct((N, H), jnp.float32),
        jax.ShapeDtypeStruct((G, H), jnp.float32),
    ]
    if has_next:
        in_specs += ([pl.BlockSpec((H, 32), lambda i: (0, 0))] * 2
                     + [pl.BlockSpec((1, 32), lambda i: (0, 0))] * 2)
        args += [w1n[:, :32], w1n[:, 32:], b1n[:, :32], b1n[:, 32:]]
        out_specs += [pl.BlockSpec((BLK, 32), lambda i: (i, 0))] * 2
        out_shape += [jax.ShapeDtypeStruct((N, 32), jnp.float32)] * 2
    return pl.pallas_call(
        _make_node_body(in_dim, has_next),
        grid=(N // BLK,),
        in_specs=in_specs,
        out_specs=out_specs,
        out_shape=out_shape,
    )(*args)


def kernel(gate_type, edge_index, edge_w, graph_id, params):
    layers = params['layers']
    emb = params['emb']

    gate2d = gate_type.reshape(N, 1)
    gid2d = graph_id.reshape(N, 1)
    src2d = edge_index[0].reshape(E // SUB, SUB)
    dst2d = edge_index[1].reshape(E // SUB, SUB)
    ewg = edge_w.T.reshape(3, E // 16, 16)
    zrows = jnp.zeros((RPT, 32), jnp.float32)

    dims = [EMBED, H, H]
    w1_1 = layers[0]['W1']
    b1_1 = layers[0]['b1'].reshape(1, H)
    h0, pa, pb = _embed(gate2d, emb, w1_1[:EMBED, :32], w1_1[:EMBED, 32:],
                        b1_1[:, :32], b1_1[:, 32:])

    h = h0
    readouts = []
    for l in range(3):
        w1e = layers[l]['W1'][dims[l]:, :]
        hna, hnb = _sc_edge(src2d, dst2d, pa, pb, ewg,
                            w1e[:, :32], w1e[:, 32:], zrows)
        w2 = layers[l]['W2']
        in_dim = dims[l]
        w1n = layers[l + 1]['W1'][:H, :] if l < 2 else None
        b1n = layers[l + 1]['b1'].reshape(1, H) if l < 2 else None
        outs = _node_update(h, hna, hnb, gid2d,
                            w2[:in_dim, :], w2[in_dim:in_dim + 32, :],
                            w2[in_dim + 32:, :], w1n, b1n)
        if l < 2:
            h, r_l, pa, pb = outs
        else:
            h, r_l = outs
        readouts.append(r_l)

    return h, jnp.concatenate(readouts, axis=1)
